# baseline (speedup 1.0000x reference)
import numpy as np
import jax
import jax.numpy as jnp
from jax import lax
from jax.experimental import pallas as pl
from jax.experimental.pallas import tpu as pltpu
from jax.experimental.pallas import tpu_sc as plsc

N_BEH = 8192
N_OBJ = 256
N_PROP = 16
STEP_DIST = 0.05
NW = 32
BEH_PER_W = N_BEH // NW
L = 16
N_GROUPS = BEH_PER_W // L

_R2 = np.float32(np.sqrt(2.0) / 2.0)
_CT = np.array([-1, -_R2, 0, _R2, 1, _R2, 0, -_R2, -1], np.float32)
_ST = np.array([0, -_R2, -1, -_R2, 0, _R2, 1, _R2, 0], np.float32)
_TAU = np.float32(np.tan(np.float64(22.5) * np.pi / 180.0))


def _dist_bin_intervals():
    u = np.float32(0.05)
    inf = np.float32(np.inf)
    los = np.zeros(21, np.float32)
    his = np.zeros(21, np.float32)

    def ok(v, k):
        return np.float32(np.round(np.float32(v) / u)) == np.float32(k)

    for k in range(21):
        v = np.float32(max((k - 0.5) * 0.05 - 2e-6, 0.0))
        while not ok(v, k):
            v = np.nextafter(v, inf, dtype=np.float32)
        los[k] = v
        v = np.float32((k + 0.5) * 0.05 + 2e-6)
        while not ok(v, k):
            v = np.nextafter(v, -inf, dtype=np.float32)
        his[k] = v
    return los, his


_LOS, _HIS = _dist_bin_intervals()

_P_P0, _P_P1, _P_MVX, _P_MVY = 0, 1, 2, 3
_P_LOX, _P_HIX, _P_LOY, _P_HIY = 4, 5, 6, 7
_P_CT, _P_ST, _P_A2, _P_B2, _P_W = 8, 9, 10, 11, 12
_NPAR = 13


def _vtake(vec, idx):
    dnums = lax.GatherDimensionNumbers(offset_dims=(), collapsed_slice_dims=(0,),
                                       start_index_map=(0,))
    return lax.gather(vec, idx[:, None], dnums, slice_sizes=(1,),
                      mode=lax.GatherScatterMode.PROMISE_IN_BOUNDS)


def _sc_body(x_hbm, par_hbm, omt_hbm, out_hbm, x_v, par_v, om_v0, om_v1,
             out_v, sem0, sem1):
    wid = lax.axis_index("s") * 2 + lax.axis_index("c")
    base = wid * SC_PER_W
    h0 = pltpu.async_copy(
        omt_hbm.at[pl.ds(0, 128), pl.ds(base, SC_PER_W)], om_v0, sem0)
    h1 = pltpu.async_copy(
        omt_hbm.at[pl.ds(128, 128), pl.ds(base, SC_PER_W)], om_v1, sem1)
    pltpu.sync_copy(x_hbm, x_v)
    pltpu.sync_copy(par_hbm.at[:, pl.ds(base, SC_PER_W)], par_v)

    def group_params(g):
        sl = pl.ds(g * L, L)
        p0v = par_v[_P_P0, sl].astype(jnp.int32)
        p1v = par_v[_P_P1, sl].astype(jnp.int32)
        xrow0 = x_v[0, :]
        pmx = _vtake(xrow0, p0v) + par_v[_P_MVX, sl]
        pmy = _vtake(xrow0, p1v) + par_v[_P_MVY, sl]
        return (sl, p0v, p1v, pmx, pmy,
                par_v[_P_LOX, sl], par_v[_P_HIX, sl],
                par_v[_P_LOY, sl], par_v[_P_HIY, sl],
                par_v[_P_CT, sl], par_v[_P_ST, sl],
                par_v[_P_A2, sl], par_v[_P_B2, sl])

    def make_body(om_half, joff, prm):
        (sl, p0v, p1v, pmx, pmy, lox, hix, loy, hiy, ct, st, a2, b2) = prm

        def body(j, acc):
            xrow = x_v[j + joff, :]
            c0 = _vtake(xrow, p0v)
            c1 = _vtake(xrow, p1v)
            omv = om_half[j, sl]
            dx = c0 - pmx
            dy = c1 - pmy
            adx = jnp.abs(dx)
            ady = jnp.abs(dy)
            sx = jnp.minimum(adx - lox, hix - adx)
            sy = jnp.minimum(ady - loy, hiy - ady)
            u = dx * ct + dy * st
            v = dy * ct - dx * st
            ssec = _TAU * u - jnp.abs(v)
            ssgn = a2 + b2 * v
            slack = jnp.minimum(jnp.minimum(sx, sy),
                                jnp.minimum(jnp.minimum(ssec, ssgn), omv))
            return jnp.maximum(acc, slack)

        return body

    h0.wait()
    for g in range(SC_GROUPS):
        prm = group_params(g)
        acc = lax.fori_loop(0, 128, make_body(om_v0, 0, prm),
                            jnp.full((L,), -1.0, jnp.float32))
        out_v[prm[0]] = acc

    h1.wait()
    for g in range(SC_GROUPS):
        prm = group_params(g)
        acc = lax.fori_loop(0, 128, make_body(om_v1, 128, prm),
                            out_v[prm[0]])
        hit = jnp.minimum(jnp.sign(acc) + 1.0, 1.0)
        out_v[prm[0]] = hit * par_v[_P_W, prm[0]]

    pltpu.sync_copy(out_v, out_hbm.at[pl.ds(base, SC_PER_W)])


def _sc_call(x2, par, omt):
    mesh = plsc.VectorSubcoreMesh(core_axis_name="c", subcore_axis_name="s")
    return pl.kernel(
        _sc_body,
        mesh=mesh,
        out_type=jax.ShapeDtypeStruct((SC_BEH,), jnp.float32),
        scratch_types=[
            pltpu.VMEM((N_OBJ, N_PROP), jnp.float32),
            pltpu.VMEM((_NPAR, SC_PER_W), jnp.float32),
            pltpu.VMEM((128, SC_PER_W), jnp.float32),
            pltpu.VMEM((128, SC_PER_W), jnp.float32),
            pltpu.VMEM((SC_PER_W,), jnp.float32),
            pltpu.SemaphoreType.DMA,
            pltpu.SemaphoreType.DMA,
        ],
    )(x2, par, omt)


H_TC = 4096
BLK = 256
SC_BEH = N_BEH - H_TC
SC_PER_W = SC_BEH // NW
SC_GROUPS = SC_PER_W // L


def _tc_body(xt_ref, p0_ref, p1_ref, mvx_ref, mvy_ref, lox_ref, hix_ref,
             loy_ref, hiy_ref, ct_ref, st_ref, s4_ref, om_ref, w_ref, out_ref):
    xt = xt_ref[...]
    p0 = p0_ref[...]
    p1 = p1_ref[...]

    prop_iota = jax.lax.broadcasted_iota(jnp.int32, (BLK, N_PROP), 1)
    oh0 = (prop_iota == p0[:, None]).astype(jnp.float32)
    oh1 = (prop_iota == p1[:, None]).astype(jnp.float32)
    c0 = jnp.dot(oh0, xt, preferred_element_type=jnp.float32,
                 precision=jax.lax.Precision.HIGHEST)
    c1 = jnp.dot(oh1, xt, preferred_element_type=jnp.float32,
                 precision=jax.lax.Precision.HIGHEST)

    p1mx = c0[:, 0] + mvx_ref[...]
    p1my = c1[:, 0] + mvy_ref[...]
    dx = c0 - p1mx[:, None]
    dy = c1 - p1my[:, None]

    adx = jnp.abs(dx)
    ady = jnp.abs(dy)
    mask_x = (lox_ref[...][:, None] <= adx) & (adx <= hix_ref[...][:, None])
    mask_y = (loy_ref[...][:, None] <= ady) & (ady <= hiy_ref[...][:, None])

    ct = ct_ref[...][:, None]
    st = st_ref[...][:, None]
    s4 = s4_ref[...][:, None]
    u = dx * ct + dy * st
    v = dy * ct - dx * st
    sector = jnp.abs(v) <= _TAU * u
    sgn_ok = ((s4 == 0.0) | ((s4 > 0.0) & (v <= 0.0))
              | ((s4 < 0.0) & (v > 0.0)))
    mask = sector & sgn_ok & mask_x & mask_y & (om_ref[...] > 0.0)
    hit = jnp.sum(mask.astype(jnp.float32), axis=1) > 0.0
    out_ref[...] = hit.astype(jnp.float32) * w_ref[...]


def _tc_call(xt, p0, p1, mvx, mvy, lox, hix, loy, hiy, ct, st, s4, omf, w):
    grid = (H_TC // BLK,)
    beh_spec = pl.BlockSpec((BLK,), lambda i: (i,))
    return pl.pallas_call(
        _tc_body,
        grid=grid,
        in_specs=[
            pl.BlockSpec((N_PROP, N_OBJ), lambda i: (0, 0)),
            beh_spec, beh_spec, beh_spec, beh_spec, beh_spec, beh_spec,
            beh_spec, beh_spec, beh_spec, beh_spec, beh_spec,
            pl.BlockSpec((BLK, N_OBJ), lambda i: (i, 0)),
            beh_spec,
        ],
        out_specs=beh_spec,
        out_shape=jax.ShapeDtypeStruct((H_TC,), jnp.float32),
    )(xt, p0, p1, mvx, mvy, lox, hix, loy, hiy, ct, st, s4, omf, w)


def kernel(x, p, move_directions, dir_types, x_types, y_types, o_mask,
           beh_weights):
    xt = x[0].T
    obj_idx = jnp.arange(N_OBJ)[None, :]
    keep = o_mask & (obj_idx > 0)

    rad = move_directions * (jnp.pi / 180.0)
    mvx = STEP_DIST * jnp.cos(rad)
    mvy = STEP_DIST * jnp.sin(rad)

    kx = jnp.round(x_types * 20.0).astype(jnp.int32)
    ky = jnp.round(y_types * 20.0).astype(jnp.int32)
    t = jnp.round(dir_types / 45.0).astype(jnp.int32)
    lox = jnp.asarray(_LOS)[kx]
    hix = jnp.asarray(_HIS)[kx]
    loy = jnp.asarray(_LOS)[ky]
    hiy = jnp.asarray(_HIS)[ky]
    ctv = jnp.asarray(_CT)[t + 4]
    stv = jnp.asarray(_ST)[t + 4]
    s4 = jnp.where(t == 4, 1.0, jnp.where(t == -4, -1.0, 0.0)).astype(jnp.float32)
    a2 = jnp.where(t == 4, 0.0, jnp.where(t == -4, -1.0, 1.0)).astype(jnp.float32)
    b2 = jnp.where(t == 4, -1.0, jnp.where(t == -4, 1e20, 0.0)).astype(jnp.float32)
    p0 = p[:, 0].astype(jnp.int32)
    p1 = p[:, 1].astype(jnp.int32)

    s = slice(H_TC, N_BEH)
    omt_sc = jnp.where(keep[s], 1.0, -1.0).astype(jnp.float32).T
    par_sc = jnp.stack([
        p0[s].astype(jnp.float32), p1[s].astype(jnp.float32),
        mvx[s], mvy[s], lox[s], hix[s], loy[s], hiy[s],
        ctv[s], stv[s], a2[s], b2[s], beh_weights[s],
    ])
    out_sc = _sc_call(x[0], par_sc, omt_sc)

    h = slice(0, H_TC)
    omf_tc = jnp.where(keep[h], 1.0, 0.0).astype(jnp.float32)
    out_tc = _tc_call(xt, p0[h], p1[h], mvx[h], mvy[h], lox[h], hix[h],
                      loy[h], hiy[h], ctv[h], stv[h], s4[h], omf_tc,
                      beh_weights[h])

    return jnp.concatenate([out_tc, out_sc])


# speedup vs baseline: 1.0190x; 1.0190x over previous
import numpy as np
import jax
import jax.numpy as jnp
from jax import lax
from jax.experimental import pallas as pl
from jax.experimental.pallas import tpu as pltpu
from jax.experimental.pallas import tpu_sc as plsc

N_BEH = 8192
N_OBJ = 256
N_PROP = 16
STEP_DIST = 0.05
NW = 32
BEH_PER_W = N_BEH // NW
L = 16
N_GROUPS = BEH_PER_W // L

_R2 = np.float32(np.sqrt(2.0) / 2.0)
_CT = np.array([-1, -_R2, 0, _R2, 1, _R2, 0, -_R2, -1], np.float32)
_ST = np.array([0, -_R2, -1, -_R2, 0, _R2, 1, _R2, 0], np.float32)
_TAU = np.float32(np.tan(np.float64(22.5) * np.pi / 180.0))


def _dist_bin_intervals():
    u = np.float32(0.05)
    inf = np.float32(np.inf)
    los = np.zeros(21, np.float32)
    his = np.zeros(21, np.float32)

    def ok(v, k):
        return np.float32(np.round(np.float32(v) / u)) == np.float32(k)

    for k in range(21):
        v = np.float32(max((k - 0.5) * 0.05 - 2e-6, 0.0))
        while not ok(v, k):
            v = np.nextafter(v, inf, dtype=np.float32)
        los[k] = v
        v = np.float32((k + 0.5) * 0.05 + 2e-6)
        while not ok(v, k):
            v = np.nextafter(v, -inf, dtype=np.float32)
        his[k] = v
    return los, his


_LOS, _HIS = _dist_bin_intervals()

_P_P0, _P_P1, _P_MVX, _P_MVY = 0, 1, 2, 3
_P_LOX, _P_HIX, _P_LOY, _P_HIY = 4, 5, 6, 7
_P_CT, _P_ST, _P_A2, _P_B2, _P_W = 8, 9, 10, 11, 12
_NPAR = 13


def _vtake(vec, idx):
    dnums = lax.GatherDimensionNumbers(offset_dims=(), collapsed_slice_dims=(0,),
                                       start_index_map=(0,))
    return lax.gather(vec, idx[:, None], dnums, slice_sizes=(1,),
                      mode=lax.GatherScatterMode.PROMISE_IN_BOUNDS)


def _sc_body(x_hbm, par_hbm, omt_hbm, out_hbm, x_v, par_v, om_v, out_v):
    wid = lax.axis_index("s") * 2 + lax.axis_index("c")
    base = wid * SC_PER_W
    pltpu.sync_copy(x_hbm, x_v)
    pltpu.sync_copy(par_hbm.at[:, pl.ds(base, SC_PER_W)], par_v)
    pltpu.sync_copy(omt_hbm.at[:, pl.ds(base, SC_PER_W)], om_v)

    for g in range(SC_GROUPS):
        sl = pl.ds(g * L, L)
        p0v = par_v[_P_P0, sl].astype(jnp.int32)
        p1v = par_v[_P_P1, sl].astype(jnp.int32)
        xrow0 = x_v[0, :]
        pmx = _vtake(xrow0, p0v) + par_v[_P_MVX, sl]
        pmy = _vtake(xrow0, p1v) + par_v[_P_MVY, sl]
        lox = par_v[_P_LOX, sl]
        hix = par_v[_P_HIX, sl]
        loy = par_v[_P_LOY, sl]
        hiy = par_v[_P_HIY, sl]
        ct = par_v[_P_CT, sl]
        st = par_v[_P_ST, sl]
        a2 = par_v[_P_A2, sl]
        b2 = par_v[_P_B2, sl]

        def body(j, acc):
            xrow = x_v[j, :]
            c0 = _vtake(xrow, p0v)
            c1 = _vtake(xrow, p1v)
            omv = om_v[j, sl]
            dx = c0 - pmx
            dy = c1 - pmy
            adx = jnp.abs(dx)
            ady = jnp.abs(dy)
            sx = jnp.minimum(adx - lox, hix - adx)
            sy = jnp.minimum(ady - loy, hiy - ady)
            u = dx * ct + dy * st
            v = dy * ct - dx * st
            ssec = _TAU * u - jnp.abs(v)
            ssgn = a2 + b2 * v
            slack = jnp.minimum(jnp.minimum(sx, sy),
                                jnp.minimum(jnp.minimum(ssec, ssgn), omv))
            return jnp.maximum(acc, slack)

        acc = lax.fori_loop(0, N_OBJ, body, jnp.full((L,), -1.0, jnp.float32))
        hit = jnp.minimum(jnp.sign(acc) + 1.0, 1.0)
        out_v[sl] = hit * par_v[_P_W, sl]

    pltpu.sync_copy(out_v, out_hbm.at[pl.ds(base, SC_PER_W)])


def _sc_call(x2, par, omt):
    mesh = plsc.VectorSubcoreMesh(core_axis_name="c", subcore_axis_name="s")
    return pl.kernel(
        _sc_body,
        mesh=mesh,
        out_type=jax.ShapeDtypeStruct((SC_BEH,), jnp.float32),
        scratch_types=[
            pltpu.VMEM((N_OBJ, N_PROP), jnp.float32),
            pltpu.VMEM((_NPAR, SC_PER_W), jnp.float32),
            pltpu.VMEM((N_OBJ, SC_PER_W), jnp.float32),
            pltpu.VMEM((SC_PER_W,), jnp.float32),
        ],
    )(x2, par, omt)


H_TC = 4096
BLK = 256
SC_BEH = N_BEH - H_TC
SC_PER_W = SC_BEH // NW
SC_GROUPS = SC_PER_W // L


def _tc_body(xt_ref, p0_ref, p1_ref, mvx_ref, mvy_ref, lox_ref, hix_ref,
             loy_ref, hiy_ref, ct_ref, st_ref, s4_ref, om_ref, w_ref, out_ref):
    xt = xt_ref[...]
    p0 = p0_ref[...]
    p1 = p1_ref[...]

    prop_iota = jax.lax.broadcasted_iota(jnp.int32, (BLK, N_PROP), 1)
    oh0 = (prop_iota == p0[:, None]).astype(jnp.float32)
    oh1 = (prop_iota == p1[:, None]).astype(jnp.float32)
    c0 = jnp.dot(oh0, xt, preferred_element_type=jnp.float32,
                 precision=jax.lax.Precision.HIGHEST)
    c1 = jnp.dot(oh1, xt, preferred_element_type=jnp.float32,
                 precision=jax.lax.Precision.HIGHEST)

    p1mx = c0[:, 0] + mvx_ref[...]
    p1my = c1[:, 0] + mvy_ref[...]
    dx = c0 - p1mx[:, None]
    dy = c1 - p1my[:, None]

    adx = jnp.abs(dx)
    ady = jnp.abs(dy)
    mask_x = (lox_ref[...][:, None] <= adx) & (adx <= hix_ref[...][:, None])
    mask_y = (loy_ref[...][:, None] <= ady) & (ady <= hiy_ref[...][:, None])

    ct = ct_ref[...][:, None]
    st = st_ref[...][:, None]
    s4 = s4_ref[...][:, None]
    u = dx * ct + dy * st
    v = dy * ct - dx * st
    sector = jnp.abs(v) <= _TAU * u
    sgn_ok = ((s4 == 0.0) | ((s4 > 0.0) & (v <= 0.0))
              | ((s4 < 0.0) & (v > 0.0)))
    mask = sector & sgn_ok & mask_x & mask_y & (om_ref[...] > 0.0)
    hit = jnp.sum(mask.astype(jnp.float32), axis=1) > 0.0
    out_ref[...] = hit.astype(jnp.float32) * w_ref[...]


def _tc_call(xt, p0, p1, mvx, mvy, lox, hix, loy, hiy, ct, st, s4, omf, w):
    grid = (H_TC // BLK,)
    beh_spec = pl.BlockSpec((BLK,), lambda i: (i,))
    return pl.pallas_call(
        _tc_body,
        grid=grid,
        in_specs=[
            pl.BlockSpec((N_PROP, N_OBJ), lambda i: (0, 0)),
            beh_spec, beh_spec, beh_spec, beh_spec, beh_spec, beh_spec,
            beh_spec, beh_spec, beh_spec, beh_spec, beh_spec,
            pl.BlockSpec((BLK, N_OBJ), lambda i: (i, 0)),
            beh_spec,
        ],
        out_specs=beh_spec,
        out_shape=jax.ShapeDtypeStruct((H_TC,), jnp.float32),
    )(xt, p0, p1, mvx, mvy, lox, hix, loy, hiy, ct, st, s4, omf, w)


def kernel(x, p, move_directions, dir_types, x_types, y_types, o_mask,
           beh_weights):
    xt = x[0].T
    obj_idx = jnp.arange(N_OBJ)[None, :]
    keep = o_mask & (obj_idx > 0)

    rad = move_directions * (jnp.pi / 180.0)
    mvx = STEP_DIST * jnp.cos(rad)
    mvy = STEP_DIST * jnp.sin(rad)

    kx = jnp.round(x_types * 20.0).astype(jnp.int32)
    ky = jnp.round(y_types * 20.0).astype(jnp.int32)
    t = jnp.round(dir_types / 45.0).astype(jnp.int32)
    lox = jnp.asarray(_LOS)[kx]
    hix = jnp.asarray(_HIS)[kx]
    loy = jnp.asarray(_LOS)[ky]
    hiy = jnp.asarray(_HIS)[ky]
    ctv = jnp.asarray(_CT)[t + 4]
    stv = jnp.asarray(_ST)[t + 4]
    s4 = jnp.where(t == 4, 1.0, jnp.where(t == -4, -1.0, 0.0)).astype(jnp.float32)
    a2 = jnp.where(t == 4, 0.0, jnp.where(t == -4, -1.0, 1.0)).astype(jnp.float32)
    b2 = jnp.where(t == 4, -1.0, jnp.where(t == -4, 1e20, 0.0)).astype(jnp.float32)
    p0 = p[:, 0].astype(jnp.int32)
    p1 = p[:, 1].astype(jnp.int32)

    s = slice(H_TC, N_BEH)
    omt_sc = jnp.where(keep[s], 1.0, -1.0).astype(jnp.float32).T
    par_sc = jnp.stack([
        p0[s].astype(jnp.float32), p1[s].astype(jnp.float32),
        mvx[s], mvy[s], lox[s], hix[s], loy[s], hiy[s],
        ctv[s], stv[s], a2[s], b2[s], beh_weights[s],
    ])
    out_sc = _sc_call(x[0], par_sc, omt_sc)

    h = slice(0, H_TC)
    omf_tc = jnp.where(keep[h], 1.0, 0.0).astype(jnp.float32)
    out_tc = _tc_call(xt, p0[h], p1[h], mvx[h], mvy[h], lox[h], hix[h],
                      loy[h], hiy[h], ctv[h], stv[h], s4[h], omf_tc,
                      beh_weights[h])

    return jnp.concatenate([out_tc, out_sc])
